# baseline (speedup 1.0000x reference)
import dataclasses
import functools

import jax
import jax.numpy as jnp
import numpy as np
from jax import lax
from jax.experimental import pallas as pl
from jax.experimental.pallas import tpu as pltpu
from jax.experimental.pallas import tpu_sc as plsc

NCORES = 2
NSUB = 16
NTILES = NCORES * NSUB
CHUNK = 128


def _dense1_body(x_ref, w_ref, asrc_ref, adst_ref, mh_ref, ma_ref,
                 tab_ref, adt_ref):
    n = x_ref.shape[0]
    npad = adt_ref.shape[0] - n
    h = jnp.dot(x_ref[...], w_ref[...], preferred_element_type=jnp.float32)
    a_s = jnp.dot(h, asrc_ref[...], preferred_element_type=jnp.float32)
    a_d = jnp.dot(h, adst_ref[...], preferred_element_type=jnp.float32)
    tab = jnp.dot(h, mh_ref[...], preferred_element_type=jnp.float32) + \
        jnp.dot(a_s, ma_ref[...], preferred_element_type=jnp.float32)
    tab_ref[...] = tab.astype(jnp.bfloat16)
    adt_ref[:, 0:8] = jnp.concatenate(
        [a_d, jnp.zeros((npad, 8), jnp.float32)], axis=0)
    adt_ref[:, 8:16] = jnp.zeros_like(adt_ref[:, 8:16])


def _mid_body(part_ref, w2_ref, as2_ref, ad2_ref, b1_ref, rep_ref,
              mh_ref, ma_ref, tab2_ref, adt2_ref):
    n = tab2_ref.shape[0]
    acc = part_ref[0, :, :] + part_ref[1, :, :]
    msg = acc[0:n, 0:64]
    den = acc[0:n, 64:72]
    den_e = jnp.dot(den, rep_ref[...], preferred_element_type=jnp.float32)
    h1 = msg / (den_e + 1e-16) + b1_ref[...]
    h1 = jnp.where(h1 >= 0.0, h1, 0.2 * h1)
    h2 = jnp.dot(h1, w2_ref[...], preferred_element_type=jnp.float32)
    a2s = jnp.dot(h2, as2_ref[...], preferred_element_type=jnp.float32)
    a2d = jnp.dot(h2, ad2_ref[...], preferred_element_type=jnp.float32)
    tab2 = jnp.dot(h2, mh_ref[...], preferred_element_type=jnp.float32) + \
        jnp.dot(a2s, ma_ref[...], preferred_element_type=jnp.float32)
    tab2_ref[...] = tab2.astype(jnp.bfloat16)
    npad = adt2_ref.shape[0] - n
    adt2_ref[:, 0:8] = jnp.zeros_like(adt2_ref[:, 0:8])
    adt2_ref[:, 8:9] = jnp.concatenate(
        [a2d, jnp.zeros((npad, 1), jnp.float32)], axis=0)
    adt2_ref[:, 9:16] = jnp.zeros_like(adt2_ref[:, 9:16])


def _final_body(part2_ref, b2_ref, out_ref):
    n, c = out_ref.shape
    acc = part2_ref[0, :, :] + part2_ref[1, :, :]
    z = acc[0:n, 0:c] / (acc[0:n, 40:41] + 1e-16) + b2_ref[...]
    m = jnp.max(z, axis=1, keepdims=True)
    z = z - m
    lse = jnp.log(jnp.sum(jnp.exp(z), axis=1, keepdims=True))
    out_ref[...] = z - lse


def _make_edge_pass(nacc, wtab, wb, ep, layer):
    per_tile = ep // NTILES
    n_chunks = per_tile // CHUNK
    rows_per_sub = nacc // NSUB
    zr = rows_per_sub // 8
    assert n_chunks % 2 == 0 and rows_per_sub % 8 == 0
    mesh = plsc.VectorSubcoreMesh(core_axis_name="c", subcore_axis_name="s",
                                  num_cores=NCORES, num_subcores=NSUB)
    cp = pltpu.CompilerParams()
    if "needs_layout_passes" in pltpu.CompilerParams.__dataclass_fields__:
        cp = dataclasses.replace(cp, needs_layout_passes=False)
    if "use_tc_tiling_on_sc" in pltpu.CompilerParams.__dataclass_fields__:
        cp = dataclasses.replace(cp, use_tc_tiling_on_sc=False)

    @functools.partial(
        pl.kernel,
        compiler_params=cp,
        out_type=jax.ShapeDtypeStruct((NCORES, nacc, wtab), jnp.float32),
        mesh=mesh,
        scratch_types=[
            pltpu.VMEM((n_chunks, CHUNK), jnp.int32),
            pltpu.VMEM((n_chunks, CHUNK), jnp.int32),
            pltpu.VMEM((CHUNK, wb), jnp.bfloat16),
            pltpu.VMEM((CHUNK, wb), jnp.bfloat16),
            pltpu.VMEM((CHUNK, 16), jnp.float32),
            pltpu.VMEM((CHUNK, 16), jnp.float32),
            pltpu.VMEM((CHUNK, wtab), jnp.float32),
            pltpu.VMEM((CHUNK, wtab), jnp.float32),
            pltpu.VMEM_SHARED((nacc, wtab), jnp.float32),
            pltpu.VMEM_SHARED((nacc, 16), jnp.float32),
            pltpu.VMEM((nacc // NSUB // 8, wtab), jnp.float32),
            pltpu.SemaphoreType.DMA,
            pltpu.SemaphoreType.DMA,
            pltpu.SemaphoreType.DMA,
            pltpu.SemaphoreType.DMA,
            pltpu.SemaphoreType.DMA,
            pltpu.SemaphoreType.DMA,
        ],
    )
    def edge_kernel(tab_hbm, adt_hbm, src_hbm, dst_hbm, out_hbm,
                    src_all, dst_all, rows0, rows1, drows0, drows1,
                    orows0, orows1, acc_sh, adt_sh, zbuf,
                    semt0, semt1, semd0, semd1, sems0, sems1):
        cid = lax.axis_index("c")
        sid = lax.axis_index("s")
        rows = (rows0, rows1)
        drows = (drows0, drows1)
        orows = (orows0, orows1)
        semt = (semt0, semt1)
        semd = (semd0, semd1)
        sems = (sems0, sems1)
        z16 = jnp.zeros((16,), jnp.float32)

        @pl.loop(0, zr)
        def _(r):
            for j in range(wtab // 16):
                zbuf[r, pl.ds(16 * j, 16)] = z16

        base_r = sid * rows_per_sub

        @pl.loop(0, 8)
        def _(k):
            pltpu.sync_copy(zbuf, acc_sh.at[pl.ds(base_r + k * zr, zr), :])

        pltpu.sync_copy(adt_hbm.at[pl.ds(base_r, rows_per_sub), :],
                        adt_sh.at[pl.ds(base_r, rows_per_sub), :])
        plsc.subcore_barrier()

        tile = cid * NSUB + sid
        io = lax.iota(jnp.int32, 16)

        pltpu.sync_copy(src_hbm.at[pl.ds(tile * n_chunks, n_chunks), :],
                        src_all)
        pltpu.sync_copy(dst_hbm.at[pl.ds(tile * n_chunks, n_chunks), :],
                        dst_all)

        def start_gathers(ch, par):
            pltpu.make_async_copy(
                tab_hbm.at[src_all.at[ch]], rows[par], semt[par]).start()
            pltpu.make_async_copy(
                adt_sh.at[dst_all.at[ch]], drows[par], semd[par]).start()

        def wait_gathers(ch, par):
            pltpu.make_async_copy(
                tab_hbm.at[src_all.at[ch]], rows[par], semt[par]).wait()
            pltpu.make_async_copy(
                adt_sh.at[dst_all.at[ch]], drows[par], semd[par]).wait()

        for par in range(2):
            start_gathers(par, par)

        @pl.loop(0, n_chunks, step=2)
        def _(ch0):
            for par in range(2):
                ch = ch0 + par
                wait_gathers(ch, par)
                @pl.when(ch0 >= 2)
                def _():
                    pltpu.make_async_copy(
                        orows[par], acc_sh.at[dst_all.at[ch - 2]],
                        sems[par]).wait()

                @pl.loop(0, CHUNK)
                def _(b):
                    a_d = drows[par][b, pl.ds(0, 16)]
                    bb = jnp.full((16,), b, jnp.int32)
                    if layer == 1:
                        a_s, _ = plsc.unpack(
                            rows[par][b, pl.ds(64, 32)],
                            format=plsc.PackFormat.INTERLEAVED)
                        e = a_s + a_d
                        e = jnp.where(e >= 0.0, e, 0.2 * e)
                        p = jnp.exp(e)
                        orows[par][b, pl.ds(64, 16)] = p
                        hi = jnp.where(io >= 8, 1, 0)
                        h01 = plsc.unpack(rows[par][b, pl.ds(0, 32)],
                                          format=plsc.PackFormat.INTERLEAVED)
                        h23 = plsc.unpack(rows[par][b, pl.ds(32, 32)],
                                          format=plsc.PackFormat.INTERLEAVED)
                        hs = (*h01, *h23)
                        for j in range(4):
                            pj = plsc.load_gather(
                                orows[par], [bb, hi + (64 + 2 * j)])
                            orows[par][b, pl.ds(16 * j, 16)] = pj * hs[j]
                    else:
                        v2, _ = plsc.unpack(
                            rows[par][b, pl.ds(32, 32)],
                            format=plsc.PackFormat.INTERLEAVED)
                        e = v2 + a_d
                        e = jnp.where(e >= 0.0, e, 0.2 * e)
                        p = jnp.exp(e)
                        orows[par][b, pl.ds(32, 16)] = p
                        pb = plsc.load_gather(orows[par], [bb, io * 0 + 40])
                        v0, v1 = plsc.unpack(
                            rows[par][b, pl.ds(0, 32)],
                            format=plsc.PackFormat.INTERLEAVED)
                        orows[par][b, pl.ds(0, 16)] = pb * v0
                        orows[par][b, pl.ds(16, 16)] = pb * v1
                        hm = jnp.where(io < 8, v2, 1.0)
                        orows[par][b, pl.ds(32, 16)] = pb * hm

                pltpu.make_async_copy(
                    orows[par], acc_sh.at[dst_all.at[ch]],
                    sems[par]).start(add=True)

                @pl.when(ch + 2 < n_chunks)
                def _():
                    start_gathers(ch + 2, par)

        for par in range(2):
            pltpu.make_async_copy(
                orows[par], acc_sh.at[dst_all.at[n_chunks - 2 + par]],
                sems[par]).wait()

        plsc.subcore_barrier()
        pltpu.sync_copy(acc_sh.at[pl.ds(base_r, rows_per_sub), :],
                        out_hbm.at[cid, pl.ds(base_r, rows_per_sub), :])

    return edge_kernel


def kernel(x, edge_index, W1, att_src1, att_dst1, b1, W2, att_src2, att_dst2,
           b2):
    n, f_in = x.shape
    heads, f_hid = att_src1.shape
    hid = heads * f_hid
    c = att_src2.shape[1]
    e = edge_index.shape[1]

    nacc = ((n + 1 + NSUB * 8 - 1) // (NSUB * 8)) * (NSUB * 8)
    grain = NTILES * CHUNK * 2
    ep = ((e + grain - 1) // grain) * grain

    rows = jnp.arange(hid)
    cols = jnp.repeat(jnp.arange(heads), f_hid)
    asrc_m = jnp.zeros((hid, heads), jnp.float32).at[rows, cols].set(
        att_src1.reshape(-1))
    adst_m = jnp.zeros((hid, heads), jnp.float32).at[rows, cols].set(
        att_dst1.reshape(-1))
    rep_m = jnp.zeros((heads, hid), jnp.float32).at[cols, rows].set(1.0)

    def _ileave(n_in, n_out, pairs):
        m = np.zeros((n_in, n_out), np.float32)
        for src_col, dst_col in pairs:
            m[src_col, dst_col] = 1.0
        return jnp.asarray(m)

    mh1 = _ileave(hid, 96, [(32 * blk + half * 16 + i, 32 * blk + 2 * i + half)
                            for blk in range(2) for half in range(2)
                            for i in range(16)])
    ma1 = _ileave(heads, 96, [(i, 64 + 2 * i) for i in range(heads)])
    mh2 = _ileave(c, 64, [(half * 16 + i, 2 * i + half)
                          for half in range(2) for i in range(16)] +
                  [(32 + i, 32 + 2 * i) for i in range(8)])
    ma2 = _ileave(1, 64, [(0, 32 + 16)])
    pad = ep - e
    src_p = jnp.concatenate(
        [edge_index[0], jnp.zeros((pad,), jnp.int32)]).reshape(-1, CHUNK)
    dst_p = jnp.concatenate(
        [edge_index[1], jnp.full((pad,), n, jnp.int32)]).reshape(-1, CHUNK)

    tab1, adt1 = pl.pallas_call(
        _dense1_body,
        out_shape=[jax.ShapeDtypeStruct((n, 96), jnp.bfloat16),
                   jax.ShapeDtypeStruct((nacc, 16), jnp.float32)],
    )(x, W1, asrc_m, adst_m, mh1, ma1)

    part1 = _make_edge_pass(nacc, 80, 96, ep, 1)(tab1, adt1, src_p, dst_p)

    tab2, adt2 = pl.pallas_call(
        _mid_body,
        out_shape=[jax.ShapeDtypeStruct((n, 64), jnp.bfloat16),
                   jax.ShapeDtypeStruct((nacc, 16), jnp.float32)],
    )(part1, W2, att_src2.reshape(c, 1), att_dst2.reshape(c, 1),
      b1.reshape(1, hid), rep_m, mh2, ma2)

    part2 = _make_edge_pass(nacc, 48, 64, ep, 2)(tab2, adt2, src_p, dst_p)

    out = pl.pallas_call(
        _final_body,
        out_shape=jax.ShapeDtypeStruct((n, c), jnp.float32),
    )(part2, b2.reshape(1, c))
    return out


# speedup vs baseline: 1.1290x; 1.1290x over previous
import dataclasses
import functools

import jax
import jax.numpy as jnp
import numpy as np
from jax import lax
from jax.experimental import pallas as pl
from jax.experimental.pallas import tpu as pltpu
from jax.experimental.pallas import tpu_sc as plsc

NCORES = 2
NSUB = 16
NTILES = NCORES * NSUB
CHUNK = 128


def _dense1_body(x_ref, w_ref, asrc_ref, adst_ref, mh_ref, ma_ref,
                 tab_ref, adt_ref):
    n = x_ref.shape[0]
    npad = adt_ref.shape[0] - n
    h = jnp.dot(x_ref[...], w_ref[...], preferred_element_type=jnp.float32)
    a_s = jnp.dot(h, asrc_ref[...], preferred_element_type=jnp.float32)
    a_d = jnp.dot(h, adst_ref[...], preferred_element_type=jnp.float32)
    tab = jnp.dot(h, mh_ref[...], preferred_element_type=jnp.float32) + \
        jnp.dot(a_s, ma_ref[...], preferred_element_type=jnp.float32)
    tab_ref[...] = tab.astype(jnp.bfloat16)
    adt_ref[:, 0:8] = jnp.concatenate(
        [a_d, jnp.zeros((npad, 8), jnp.float32)], axis=0)
    adt_ref[:, 8:16] = jnp.zeros_like(adt_ref[:, 8:16])


def _mid_body(part_ref, w2_ref, as2_ref, ad2_ref, b1_ref, rep_ref,
              mh_ref, ma_ref, tab2_ref, asf_ref, adf_ref):
    n = tab2_ref.shape[0]
    acc = part_ref[0, :, :] + part_ref[1, :, :]
    msg = acc[0:n, 0:64]
    den = acc[0:n, 64:72]
    den_e = jnp.dot(den, rep_ref[...], preferred_element_type=jnp.float32)
    h1 = msg / (den_e + 1e-16) + b1_ref[...]
    h1 = jnp.where(h1 >= 0.0, h1, 0.2 * h1)
    h2 = jnp.dot(h1, w2_ref[...], preferred_element_type=jnp.float32)
    a2s = jnp.dot(h2, as2_ref[...], preferred_element_type=jnp.float32)
    a2d = jnp.dot(h2, ad2_ref[...], preferred_element_type=jnp.float32)
    tab2 = jnp.dot(h2, mh_ref[...], preferred_element_type=jnp.float32) + \
        jnp.dot(a2s, ma_ref[...], preferred_element_type=jnp.float32)
    tab2_ref[...] = tab2.astype(jnp.bfloat16)
    npad = asf_ref.shape[0] - n
    zpad = jnp.zeros((npad, 1), jnp.float32)
    asf_ref[...] = jnp.concatenate([a2s, zpad], axis=0)
    adf_ref[...] = jnp.concatenate([a2d, zpad], axis=0)


def _final_body(part2_ref, b2_ref, out_ref):
    n, c = out_ref.shape
    acc = part2_ref[0, :, :] + part2_ref[1, :, :]
    z = acc[0:n, 0:c] / (acc[0:n, 40:41] + 1e-16) + b2_ref[...]
    m = jnp.max(z, axis=1, keepdims=True)
    z = z - m
    lse = jnp.log(jnp.sum(jnp.exp(z), axis=1, keepdims=True))
    out_ref[...] = z - lse


def _make_edge_pass(nacc, wtab, wb, ep, layer):
    per_tile = ep // NTILES
    n_chunks = per_tile // CHUNK
    rows_per_sub = nacc // NSUB
    zr = rows_per_sub // 8
    assert n_chunks % 2 == 0 and rows_per_sub % 8 == 0
    mesh = plsc.VectorSubcoreMesh(core_axis_name="c", subcore_axis_name="s",
                                  num_cores=NCORES, num_subcores=NSUB)
    cp = pltpu.CompilerParams()
    if "needs_layout_passes" in pltpu.CompilerParams.__dataclass_fields__:
        cp = dataclasses.replace(cp, needs_layout_passes=False)
    if "use_tc_tiling_on_sc" in pltpu.CompilerParams.__dataclass_fields__:
        cp = dataclasses.replace(cp, use_tc_tiling_on_sc=False)

    @functools.partial(
        pl.kernel,
        compiler_params=cp,
        out_type=jax.ShapeDtypeStruct((NCORES, nacc, wtab), jnp.float32),
        mesh=mesh,
        scratch_types=[
            pltpu.VMEM((n_chunks, CHUNK), jnp.int32),
            pltpu.VMEM((n_chunks, CHUNK), jnp.int32),
            pltpu.VMEM((CHUNK, wb), jnp.bfloat16),
            pltpu.VMEM((CHUNK, wb), jnp.bfloat16),
            pltpu.VMEM((CHUNK, 16), jnp.float32),
            pltpu.VMEM((CHUNK, 16), jnp.float32),
            pltpu.VMEM((CHUNK, wtab), jnp.float32),
            pltpu.VMEM((CHUNK, wtab), jnp.float32),
            pltpu.VMEM_SHARED((nacc, wtab), jnp.float32),
            pltpu.VMEM((nacc // NSUB // 8, wtab), jnp.float32),
            pltpu.SemaphoreType.DMA,
            pltpu.SemaphoreType.DMA,
            pltpu.SemaphoreType.DMA,
            pltpu.SemaphoreType.DMA,
            pltpu.SemaphoreType.DMA,
            pltpu.SemaphoreType.DMA,
        ],
    )
    def edge_kernel(tab_hbm, adt_hbm, src_hbm, dst_hbm, out_hbm,
                    src_all, dst_all, rows0, rows1, drows0, drows1,
                    orows0, orows1, acc_sh, zbuf,
                    semt0, semt1, semd0, semd1, sems0, sems1):
        cid = lax.axis_index("c")
        sid = lax.axis_index("s")
        rows = (rows0, rows1)
        drows = (drows0, drows1)
        orows = (orows0, orows1)
        semt = (semt0, semt1)
        semd = (semd0, semd1)
        sems = (sems0, sems1)
        z16 = jnp.zeros((16,), jnp.float32)

        @pl.loop(0, zr)
        def _(r):
            for j in range(wtab // 16):
                zbuf[r, pl.ds(16 * j, 16)] = z16

        base_r = sid * rows_per_sub

        @pl.loop(0, 8)
        def _(k):
            pltpu.sync_copy(zbuf, acc_sh.at[pl.ds(base_r + k * zr, zr), :])

        plsc.subcore_barrier()

        tile = cid * NSUB + sid
        io = lax.iota(jnp.int32, 16)

        pltpu.sync_copy(src_hbm.at[pl.ds(tile * n_chunks, n_chunks), :],
                        src_all)
        pltpu.sync_copy(dst_hbm.at[pl.ds(tile * n_chunks, n_chunks), :],
                        dst_all)

        def start_gathers(ch, par):
            pltpu.make_async_copy(
                tab_hbm.at[src_all.at[ch]], rows[par], semt[par]).start()
            pltpu.make_async_copy(
                adt_hbm.at[dst_all.at[ch]], drows[par], semd[par]).start()

        def wait_gathers(ch, par):
            pltpu.make_async_copy(
                tab_hbm.at[src_all.at[ch]], rows[par], semt[par]).wait()
            pltpu.make_async_copy(
                adt_hbm.at[dst_all.at[ch]], drows[par], semd[par]).wait()

        for par in range(2):
            start_gathers(par, par)

        @pl.loop(0, n_chunks, step=2)
        def _(ch0):
            for par in range(2):
                ch = ch0 + par
                wait_gathers(ch, par)
                @pl.when(ch0 >= 2)
                def _():
                    pltpu.make_async_copy(
                        orows[par], acc_sh.at[dst_all.at[ch - 2]],
                        sems[par]).wait()

                @pl.loop(0, CHUNK)
                def _(b):
                    a_d = drows[par][b, pl.ds(0, 16)]
                    bb = jnp.full((16,), b, jnp.int32)
                    if layer == 1:
                        a_s, _ = plsc.unpack(
                            rows[par][b, pl.ds(64, 32)],
                            format=plsc.PackFormat.INTERLEAVED)
                        e = a_s + a_d
                        e = jnp.where(e >= 0.0, e, 0.2 * e)
                        p = jnp.exp(e)
                        orows[par][b, pl.ds(64, 16)] = p
                        hi = jnp.where(io >= 8, 1, 0)
                        h01 = plsc.unpack(rows[par][b, pl.ds(0, 32)],
                                          format=plsc.PackFormat.INTERLEAVED)
                        h23 = plsc.unpack(rows[par][b, pl.ds(32, 32)],
                                          format=plsc.PackFormat.INTERLEAVED)
                        hs = (*h01, *h23)
                        for j in range(4):
                            pj = plsc.load_gather(
                                orows[par], [bb, hi + (64 + 2 * j)])
                            orows[par][b, pl.ds(16 * j, 16)] = pj * hs[j]
                    else:
                        v2, _ = plsc.unpack(
                            rows[par][b, pl.ds(32, 32)],
                            format=plsc.PackFormat.INTERLEAVED)
                        e = v2 + a_d
                        e = jnp.where(e >= 0.0, e, 0.2 * e)
                        p = jnp.exp(e)
                        orows[par][b, pl.ds(32, 16)] = p
                        pb = plsc.load_gather(orows[par], [bb, io * 0 + 40])
                        v0, v1 = plsc.unpack(
                            rows[par][b, pl.ds(0, 32)],
                            format=plsc.PackFormat.INTERLEAVED)
                        orows[par][b, pl.ds(0, 16)] = pb * v0
                        orows[par][b, pl.ds(16, 16)] = pb * v1
                        hm = jnp.where(io < 8, v2, 1.0)
                        orows[par][b, pl.ds(32, 16)] = pb * hm

                pltpu.make_async_copy(
                    orows[par], acc_sh.at[dst_all.at[ch]],
                    sems[par]).start(add=True)

                @pl.when(ch + 2 < n_chunks)
                def _():
                    start_gathers(ch + 2, par)

        for par in range(2):
            pltpu.make_async_copy(
                orows[par], acc_sh.at[dst_all.at[n_chunks - 2 + par]],
                sems[par]).wait()

        plsc.subcore_barrier()
        pltpu.sync_copy(acc_sh.at[pl.ds(base_r, rows_per_sub), :],
                        out_hbm.at[cid, pl.ds(base_r, rows_per_sub), :])

    return edge_kernel


def _make_edge_pass2(nacc, wtab, wb, ep):
    per_tile = ep // NTILES
    n_chunks = per_tile // CHUNK
    rows_per_sub = nacc // NSUB
    zr = rows_per_sub // 8
    assert n_chunks % 2 == 0 and rows_per_sub % 8 == 0
    mesh = plsc.VectorSubcoreMesh(core_axis_name="c", subcore_axis_name="s",
                                  num_cores=NCORES, num_subcores=NSUB)
    cp = pltpu.CompilerParams()
    if "needs_layout_passes" in pltpu.CompilerParams.__dataclass_fields__:
        cp = dataclasses.replace(cp, needs_layout_passes=False)
    if "use_tc_tiling_on_sc" in pltpu.CompilerParams.__dataclass_fields__:
        cp = dataclasses.replace(cp, use_tc_tiling_on_sc=False)

    @functools.partial(
        pl.kernel,
        compiler_params=cp,
        out_type=jax.ShapeDtypeStruct((NCORES, nacc, wtab), jnp.float32),
        mesh=mesh,
        scratch_types=[
            pltpu.VMEM((n_chunks, CHUNK), jnp.int32),
            pltpu.VMEM((n_chunks, CHUNK), jnp.int32),
            pltpu.VMEM((CHUNK, wb), jnp.bfloat16),
            pltpu.VMEM((CHUNK, wb), jnp.bfloat16),
            pltpu.VMEM((CHUNK, wtab), jnp.float32),
            pltpu.VMEM((CHUNK, wtab), jnp.float32),
            pltpu.VMEM((nacc // CHUNK, CHUNK), jnp.float32),
            pltpu.VMEM((nacc // CHUNK, CHUNK), jnp.float32),
            pltpu.VMEM((CHUNK,), jnp.float32),
            pltpu.VMEM_SHARED((nacc, wtab), jnp.float32),
            pltpu.VMEM((nacc // NSUB // 8, wtab), jnp.float32),
            pltpu.SemaphoreType.DMA,
            pltpu.SemaphoreType.DMA,
            pltpu.SemaphoreType.DMA,
            pltpu.SemaphoreType.DMA,
        ],
    )
    def edge_kernel(tab_hbm, asf_hbm, adf_hbm, src_hbm, dst_hbm, out_hbm,
                    src_all, dst_all, rows0, rows1, orows0, orows1,
                    asf_v, adf_v, pbuf, acc_sh, zbuf,
                    semt0, semt1, sems0, sems1):
        cid = lax.axis_index("c")
        sid = lax.axis_index("s")
        rows = (rows0, rows1)
        orows = (orows0, orows1)
        semt = (semt0, semt1)
        sems = (sems0, sems1)
        z16 = jnp.zeros((16,), jnp.float32)

        @pl.loop(0, zr)
        def _(r):
            for j in range(wtab // 16):
                zbuf[r, pl.ds(16 * j, 16)] = z16

        base_r = sid * rows_per_sub

        @pl.loop(0, 8)
        def _(k):
            pltpu.sync_copy(zbuf, acc_sh.at[pl.ds(base_r + k * zr, zr), :])

        pltpu.sync_copy(asf_hbm, asf_v)
        pltpu.sync_copy(adf_hbm, adf_v)
        plsc.subcore_barrier()

        tile = cid * NSUB + sid
        io = lax.iota(jnp.int32, 16)
        zi = io * 0

        pltpu.sync_copy(src_hbm.at[pl.ds(tile * n_chunks, n_chunks), :],
                        src_all)
        pltpu.sync_copy(dst_hbm.at[pl.ds(tile * n_chunks, n_chunks), :],
                        dst_all)

        def start_gathers(ch, par):
            pltpu.make_async_copy(
                tab_hbm.at[src_all.at[ch]], rows[par], semt[par]).start()

        for par in range(2):
            start_gathers(par, par)

        @pl.loop(0, n_chunks, step=2)
        def _(ch0):
            for par in range(2):
                ch = ch0 + par
                pltpu.make_async_copy(
                    tab_hbm.at[src_all.at[ch]], rows[par], semt[par]).wait()

                @pl.when(ch0 >= 2)
                def _():
                    pltpu.make_async_copy(
                        orows[par], acc_sh.at[dst_all.at[ch - 2]],
                        sems[par]).wait()

                @pl.loop(0, CHUNK // 16)
                def _(g):
                    svec = src_all[ch, pl.ds(16 * g, 16)]
                    dvec = dst_all[ch, pl.ds(16 * g, 16)]
                    a_s = plsc.load_gather(asf_v, [svec >> 7, svec & 127])
                    a_d = plsc.load_gather(adf_v, [dvec >> 7, dvec & 127])
                    e = a_s + a_d
                    e = jnp.where(e >= 0.0, e, 0.2 * e)
                    pbuf[pl.ds(16 * g, 16)] = jnp.exp(e)

                @pl.loop(0, CHUNK)
                def _(b):
                    bb = jnp.full((16,), b, jnp.int32)
                    pb = plsc.load_gather(pbuf, [bb])
                    v0, v1 = plsc.unpack(
                        rows[par][b, pl.ds(0, 32)],
                        format=plsc.PackFormat.INTERLEAVED)
                    v2, _ = plsc.unpack(
                        rows[par][b, pl.ds(32, 32)],
                        format=plsc.PackFormat.INTERLEAVED)
                    orows[par][b, pl.ds(0, 16)] = pb * v0
                    orows[par][b, pl.ds(16, 16)] = pb * v1
                    hm = jnp.where(io < 8, v2, 1.0)
                    orows[par][b, pl.ds(32, 16)] = pb * hm

                pltpu.make_async_copy(
                    orows[par], acc_sh.at[dst_all.at[ch]],
                    sems[par]).start(add=True)

                @pl.when(ch + 2 < n_chunks)
                def _():
                    start_gathers(ch + 2, par)

        for par in range(2):
            pltpu.make_async_copy(
                orows[par], acc_sh.at[dst_all.at[n_chunks - 2 + par]],
                sems[par]).wait()

        plsc.subcore_barrier()
        pltpu.sync_copy(acc_sh.at[pl.ds(base_r, rows_per_sub), :],
                        out_hbm.at[cid, pl.ds(base_r, rows_per_sub), :])

    return edge_kernel


def kernel(x, edge_index, W1, att_src1, att_dst1, b1, W2, att_src2, att_dst2,
           b2):
    n, f_in = x.shape
    heads, f_hid = att_src1.shape
    hid = heads * f_hid
    c = att_src2.shape[1]
    e = edge_index.shape[1]

    nacc = ((n + 1 + NSUB * 8 - 1) // (NSUB * 8)) * (NSUB * 8)
    grain = NTILES * CHUNK * 2
    ep = ((e + grain - 1) // grain) * grain

    rows = jnp.arange(hid)
    cols = jnp.repeat(jnp.arange(heads), f_hid)
    asrc_m = jnp.zeros((hid, heads), jnp.float32).at[rows, cols].set(
        att_src1.reshape(-1))
    adst_m = jnp.zeros((hid, heads), jnp.float32).at[rows, cols].set(
        att_dst1.reshape(-1))
    rep_m = jnp.zeros((heads, hid), jnp.float32).at[cols, rows].set(1.0)

    def _ileave(n_in, n_out, pairs):
        m = np.zeros((n_in, n_out), np.float32)
        for src_col, dst_col in pairs:
            m[src_col, dst_col] = 1.0
        return jnp.asarray(m)

    mh1 = _ileave(hid, 96, [(32 * blk + half * 16 + i, 32 * blk + 2 * i + half)
                            for blk in range(2) for half in range(2)
                            for i in range(16)])
    ma1 = _ileave(heads, 96, [(i, 64 + 2 * i) for i in range(heads)])
    mh2 = _ileave(c, 64, [(half * 16 + i, 2 * i + half)
                          for half in range(2) for i in range(16)] +
                  [(32 + i, 32 + 2 * i) for i in range(8)])
    ma2 = _ileave(1, 64, [(0, 32 + 16)])
    pad = ep - e
    src_p = jnp.concatenate(
        [edge_index[0], jnp.zeros((pad,), jnp.int32)]).reshape(-1, CHUNK)
    dst_p = jnp.concatenate(
        [edge_index[1], jnp.full((pad,), n, jnp.int32)]).reshape(-1, CHUNK)

    tab1, adt1 = pl.pallas_call(
        _dense1_body,
        out_shape=[jax.ShapeDtypeStruct((n, 96), jnp.bfloat16),
                   jax.ShapeDtypeStruct((nacc, 16), jnp.float32)],
    )(x, W1, asrc_m, adst_m, mh1, ma1)

    part1 = _make_edge_pass(nacc, 80, 96, ep, 1)(tab1, adt1, src_p, dst_p)

    tab2, asf, adf = pl.pallas_call(
        _mid_body,
        out_shape=[jax.ShapeDtypeStruct((n, 64), jnp.bfloat16),
                   jax.ShapeDtypeStruct((nacc, 1), jnp.float32),
                   jax.ShapeDtypeStruct((nacc, 1), jnp.float32)],
    )(part1, W2, att_src2.reshape(c, 1), att_dst2.reshape(c, 1),
      b1.reshape(1, hid), rep_m, mh2, ma2)

    part2 = _make_edge_pass2(nacc, 48, 64, ep)(
        tab2, asf.reshape(nacc // CHUNK, CHUNK),
        adf.reshape(nacc // CHUNK, CHUNK), src_p, dst_p)

    out = pl.pallas_call(
        _final_body,
        out_shape=jax.ShapeDtypeStruct((n, c), jnp.float32),
    )(part2, b2.reshape(1, c))
    return out
